# Initial kernel scaffold; baseline (speedup 1.0000x reference)
import jax
import jax.numpy as jnp
from jax.experimental import pallas as pl


def kernel(x, edge_index, w_conv_self, w_conv_nbr, b_conv, w_res, b_res, w_fc1, b_fc1, w_fc2, b_fc2):
    raise NotImplementedError("write your pallas kernel here")


# speedup vs baseline: 17.0551x; 17.0551x over previous
import functools

import jax
import jax.numpy as jnp
import numpy as np
from jax import lax
from jax.experimental import pallas as pl
from jax.experimental.pallas import tpu as pltpu
from jax.experimental.pallas import tpu_sc as plsc

N = 100000
E = 1600000
NF = 32

NROWS = 32
EPR = E // NROWS
CPR = 416
RPAD = CPR * 128 - EPR
NACC = N + 96
TPT = NACC // 16
WB0 = 6256
WB15 = N - 15 * WB0

_rr, _cc = np.meshgrid(np.arange(NROWS), np.arange(RPAD), indexing="ij")
_PAD_SRC = np.asarray((_rr * 1009 + _cc * 257) % N, dtype=np.int32)
_PAD_DST = np.asarray(N + (_cc % 96), dtype=np.int32)

_MESH = plsc.VectorSubcoreMesh(
    core_axis_name="c", subcore_axis_name="s", num_cores=2, num_subcores=16
)

_F32 = jnp.float32


def _make_sc_aggregate(round0: bool):
  @functools.partial(
      pl.kernel,
      out_type=[jax.ShapeDtypeStruct((N, 16), _F32),
                jax.ShapeDtypeStruct((N, 16), _F32)],
      mesh=_MESH,
      scratch_types=[
          pltpu.VMEM_SHARED((NACC, 16), _F32),
          pltpu.VMEM((4, 8, 128), jnp.int32),
          pltpu.VMEM((4, 8, 128), jnp.int32),
          pltpu.VMEM((2, 4, 128, 16), _F32),
          pltpu.SemaphoreType.DMA((2, 4)),
          pltpu.SemaphoreType.DMA((4, 2)),
      ],
      compiler_params=pltpu.CompilerParams(use_tc_tiling_on_sc=False),
  )
  def sc_agg(t0, t1, srcp, dstp, zer, out0, out1,
             acc, src_st, dst_st, rowsbuf, gsem, ssem):
    c = lax.axis_index("c")
    t = lax.axis_index("s")

    zb = t * TPT
    pltpu.sync_copy(zer.at[pl.ds(zb, TPT)], acc.at[pl.ds(zb, TPT)])
    plsc.subcore_barrier()

    if round0:
      ppr = CPR // 16
      cbase = c * (CPR // 2)
    else:
      ppr = CPR // 8
      cbase = c * 0
    npairs = 2 * ppr
    ngroups = 2 * npairs
    niter = ngroups // 8

    def stage(P, ss):
      row = 2 * t + P // ppr
      coff = cbase + (P % ppr) * 8
      pltpu.async_copy(srcp.at[row, pl.ds(coff, 8)], src_st.at[ss],
                       ssem.at[ss, 0])
      pltpu.async_copy(dstp.at[row, pl.ds(coff, 8)], dst_st.at[ss],
                       ssem.at[ss, 1])

    def wait_stage(ss):
      pltpu.make_async_copy(srcp.at[0, pl.ds(0, 8)], src_st.at[ss],
                            ssem.at[ss, 0]).wait()
      pltpu.make_async_copy(dstp.at[0, pl.ds(0, 8)], dst_st.at[ss],
                            ssem.at[ss, 1]).wait()

    def run_pipeline(table):
      def fire(gs, ss, half):
        for j in range(4):
          pltpu.async_copy(table.at[src_st.at[ss, half * 4 + j]],
                           rowsbuf.at[gs, j], gsem.at[gs, j])

      def drain(gs, ss, half):
        for j in range(4):
          pltpu.make_async_copy(table.at[src_st.at[ss, half * 4 + j]],
                                rowsbuf.at[gs, j], gsem.at[gs, j]).wait()
          pltpu.sync_copy(rowsbuf.at[gs, j], acc.at[dst_st.at[ss, half * 4 + j]],
                          add=True)

      for p in range(4):
        stage(jnp.int32(p), p)

      def loop_body(m, carry):
        for cc in range(8):
          ss = cc // 2
          half = cc % 2
          gs = cc % 2
          if half == 0:
            wait_stage(ss)
          fire(gs, ss, half)
          pgs = (cc + 1) % 2
          pss = ((cc - 1) % 8) // 2
          phalf = (cc - 1) % 2
          if cc == 0:
            @pl.when(m > 0)
            def _():
              drain(pgs, pss, phalf)

            @pl.when(m > 0)
            def _():
              stage(m * 4 + 3, 3)
          else:
            drain(pgs, pss, phalf)
            if cc in (2, 4, 6):
              rs = cc // 2 - 1
              P = m * 4 + 4 + rs

              @pl.when(P < npairs)
              def _():
                stage(P, rs)
        return carry

      lax.fori_loop(0, niter, loop_body, jnp.int32(0))
      drain(1, 3, 1)

    @pl.when(c == 0)
    def _():
      run_pipeline(t0)

    @pl.when(c == 1)
    def _():
      run_pipeline(t1)

    plsc.subcore_barrier()

    ob = t * WB0

    @pl.when(t < 15)
    def _():
      @pl.when(c == 0)
      def _():
        pltpu.sync_copy(acc.at[pl.ds(ob, WB0)], out0.at[pl.ds(ob, WB0)])

      @pl.when(c == 1)
      def _():
        pltpu.sync_copy(acc.at[pl.ds(ob, WB0)], out1.at[pl.ds(ob, WB0)])

    @pl.when(t == 15)
    def _():
      @pl.when(c == 0)
      def _():
        pltpu.sync_copy(acc.at[pl.ds(15 * WB0, WB15)],
                        out0.at[pl.ds(15 * WB0, WB15)])

      @pl.when(c == 1)
      def _():
        pltpu.sync_copy(acc.at[pl.ds(15 * WB0, WB15)],
                        out1.at[pl.ds(15 * WB0, WB15)])

  return sc_agg


_sc_agg_round0 = _make_sc_aggregate(True)
_sc_agg_rounds = _make_sc_aggregate(False)

_BLK = 2000
_NBLK = N // _BLK


def _dot_hi(a, b):
  return jax.lax.dot_general(a, b, (((1,), (0,)), ((), ())),
                             precision=jax.lax.Precision.HIGHEST,
                             preferred_element_type=_F32)


def _dot_def(a, b):
  return jax.lax.dot_general(a, b, (((1,), (0,)), ((), ())),
                             precision=jax.lax.Precision.DEFAULT,
                             preferred_element_type=_F32)


def _rt16(v):
  return v.astype(jnp.bfloat16).astype(_F32)


def _tc_combine0(x16, a0, a1, ws16, wn16r, bconv):
  def body(x_ref, a0_ref, a1_ref, ws_ref, wn_ref, b_ref,
           lo_ref, hi_ref, glo_ref, ghi_ref):
    agg = a0_ref[...] + a1_ref[...]
    h = (_dot_def(x_ref[...], ws_ref[...]) + _dot_hi(agg, wn_ref[...])
         + b_ref[...])
    h = jnp.maximum(h, 0.0)
    lo_ref[...] = h[:, :16]
    hi_ref[...] = h[:, 16:]
    g = h.astype(jnp.bfloat16).astype(_F32)
    glo_ref[...] = g[:, :16]
    ghi_ref[...] = g[:, 16:]

  return pl.pallas_call(
      body,
      grid=(_NBLK,),
      in_specs=[
          pl.BlockSpec((_BLK, 16), lambda i: (i, 0)),
          pl.BlockSpec((_BLK, 16), lambda i: (i, 0)),
          pl.BlockSpec((_BLK, 16), lambda i: (i, 0)),
          pl.BlockSpec((16, NF), lambda i: (0, 0)),
          pl.BlockSpec((16, NF), lambda i: (0, 0)),
          pl.BlockSpec((1, NF), lambda i: (0, 0)),
      ],
      out_specs=[
          pl.BlockSpec((_BLK, 16), lambda i: (i, 0)),
          pl.BlockSpec((_BLK, 16), lambda i: (i, 0)),
          pl.BlockSpec((_BLK, 16), lambda i: (i, 0)),
          pl.BlockSpec((_BLK, 16), lambda i: (i, 0)),
      ],
      out_shape=[jax.ShapeDtypeStruct((N, 16), _F32)] * 4,
  )(x16, a0, a1, ws16, wn16r, bconv)


def _tc_res(hlo, hhi, alo, ahi, w, b):
  def body(lo_ref, hi_ref, alo_ref, ahi_ref, w_ref, b_ref,
           olo_ref, ohi_ref, glo_ref, ghi_ref):
    h = jnp.concatenate([lo_ref[...], hi_ref[...]], axis=1)
    agg = jnp.concatenate([alo_ref[...], ahi_ref[...]], axis=1)
    hn = jnp.maximum(h + _dot_hi(agg, w_ref[...]) + b_ref[...], 0.0)
    olo_ref[...] = hn[:, :16]
    ohi_ref[...] = hn[:, 16:]
    g = hn.astype(jnp.bfloat16).astype(_F32)
    glo_ref[...] = g[:, :16]
    ghi_ref[...] = g[:, 16:]

  return pl.pallas_call(
      body,
      grid=(_NBLK,),
      in_specs=[
          pl.BlockSpec((_BLK, 16), lambda i: (i, 0)),
          pl.BlockSpec((_BLK, 16), lambda i: (i, 0)),
          pl.BlockSpec((_BLK, 16), lambda i: (i, 0)),
          pl.BlockSpec((_BLK, 16), lambda i: (i, 0)),
          pl.BlockSpec((NF, NF), lambda i: (0, 0)),
          pl.BlockSpec((1, NF), lambda i: (0, 0)),
      ],
      out_specs=[
          pl.BlockSpec((_BLK, 16), lambda i: (i, 0)),
          pl.BlockSpec((_BLK, 16), lambda i: (i, 0)),
          pl.BlockSpec((_BLK, 16), lambda i: (i, 0)),
          pl.BlockSpec((_BLK, 16), lambda i: (i, 0)),
      ],
      out_shape=[jax.ShapeDtypeStruct((N, 16), _F32)] * 4,
  )(hlo, hhi, alo, ahi, w, b)


def _tc_final(hlo, hhi, alo, ahi, w, b, wfc1, bfc1, wfc2, bfc2):
  def body(lo_ref, hi_ref, alo_ref, ahi_ref, w_ref, b_ref,
           w1_ref, b1_ref, w2_ref, b2_ref, out_ref):
    i = pl.program_id(0)
    h = jnp.concatenate([lo_ref[...], hi_ref[...]], axis=1)
    agg = jnp.concatenate([alo_ref[...], ahi_ref[...]], axis=1)
    h3 = jnp.maximum(h + _dot_hi(agg, w_ref[...]) + b_ref[...], 0.0)
    z = jnp.maximum(_dot_def(h3, w1_ref[...]) + b1_ref[...], 0.0)
    s = jnp.sum(_dot_def(z, w2_ref[...]))

    @pl.when(i == 0)
    def _():
      out_ref[...] = jnp.zeros((1, 1), _F32)

    out_ref[...] += s.reshape(1, 1)

    @pl.when(i == _NBLK - 1)
    def _():
      out_ref[...] = out_ref[...] / N + b2_ref[...]

  return pl.pallas_call(
      body,
      grid=(_NBLK,),
      in_specs=[
          pl.BlockSpec((_BLK, 16), lambda i: (i, 0)),
          pl.BlockSpec((_BLK, 16), lambda i: (i, 0)),
          pl.BlockSpec((_BLK, 16), lambda i: (i, 0)),
          pl.BlockSpec((_BLK, 16), lambda i: (i, 0)),
          pl.BlockSpec((NF, NF), lambda i: (0, 0)),
          pl.BlockSpec((1, NF), lambda i: (0, 0)),
          pl.BlockSpec((NF, 30), lambda i: (0, 0)),
          pl.BlockSpec((1, 30), lambda i: (0, 0)),
          pl.BlockSpec((30, 1), lambda i: (0, 0)),
          pl.BlockSpec((1, 1), lambda i: (0, 0)),
      ],
      out_specs=pl.BlockSpec((1, 1), lambda i: (0, 0)),
      out_shape=jax.ShapeDtypeStruct((1, 1), _F32),
  )(hlo, hhi, alo, ahi, w, b, wfc1, bfc1, wfc2, bfc2)


def kernel(x, edge_index, w_conv_self, w_conv_nbr, b_conv, w_res, b_res,
           w_fc1, b_fc1, w_fc2, b_fc2):
  src = edge_index[0].reshape(NROWS, EPR)
  dst = edge_index[1].reshape(NROWS, EPR)
  srcp = jnp.concatenate([src, jnp.asarray(_PAD_SRC)],
                         axis=1).reshape(NROWS, CPR, 128)
  dstp = jnp.concatenate([dst, jnp.asarray(_PAD_DST)],
                         axis=1).reshape(NROWS, CPR, 128)

  x16 = jnp.concatenate([x, jnp.zeros((N, 9), _F32)], axis=1)
  ws16 = jnp.concatenate([w_conv_self, jnp.zeros((9, NF), _F32)], axis=0)
  wn16r = _rt16(jnp.concatenate([w_conv_nbr, jnp.zeros((9, NF), _F32)],
                                axis=0))
  zer = jnp.zeros((NACC, 16), _F32)

  a0, a1 = _sc_agg_round0(_rt16(x16), _rt16(x16), srcp, dstp, zer)
  hlo, hhi, glo, ghi = _tc_combine0(x16, a0, a1, ws16, wn16r,
                                    b_conv.reshape(1, NF))

  for i in range(2):
    alo, ahi = _sc_agg_rounds(glo, ghi, srcp, dstp, zer)
    hlo, hhi, glo, ghi = _tc_res(hlo, hhi, alo, ahi, _rt16(w_res[i]),
                                 b_res[i].reshape(1, NF))

  alo, ahi = _sc_agg_rounds(glo, ghi, srcp, dstp, zer)
  out = _tc_final(hlo, hhi, alo, ahi, _rt16(w_res[2]),
                  b_res[2].reshape(1, NF), w_fc1, b_fc1.reshape(1, 30),
                  w_fc2, b_fc2.reshape(1, 1))
  return out[0, 0]


# speedup vs baseline: 26.3434x; 1.5446x over previous
import functools

import jax
import jax.numpy as jnp
import numpy as np
from jax import lax
from jax.experimental import pallas as pl
from jax.experimental.pallas import tpu as pltpu
from jax.experimental.pallas import tpu_sc as plsc

N = 100000
E = 1600000
NF = 32

NROWS = 32
EPR = E // NROWS
CPR = 416
RPAD = CPR * 128 - EPR
NACC = N + 96
TPT = NACC // 16
WB0 = 6256
WB15 = N - 15 * WB0

_rr, _cc = np.meshgrid(np.arange(NROWS), np.arange(RPAD), indexing="ij")
_PAD_SRC = np.asarray((_rr * 1009 + _cc * 257) % N, dtype=np.int32)
_PAD_DST = np.asarray(N + (_cc % 96), dtype=np.int32)

_MESH = plsc.VectorSubcoreMesh(
    core_axis_name="c", subcore_axis_name="s", num_cores=2, num_subcores=16
)

_F32 = jnp.float32


def _make_sc_aggregate(round0: bool):
  @functools.partial(
      pl.kernel,
      out_type=[jax.ShapeDtypeStruct((N, 16), _F32),
                jax.ShapeDtypeStruct((N, 16), _F32)],
      mesh=_MESH,
      scratch_types=[
          pltpu.VMEM_SHARED((NACC, 16), _F32),
          pltpu.VMEM((4, 8, 128), jnp.int32),
          pltpu.VMEM((4, 8, 128), jnp.int32),
          pltpu.VMEM((2, 4, 128, 16), _F32),
          pltpu.SemaphoreType.DMA((2, 4)),
          pltpu.SemaphoreType.DMA((4, 2)),
      ],
      compiler_params=pltpu.CompilerParams(use_tc_tiling_on_sc=False),
  )
  def sc_agg(t0, t1, srcp, dstp, zer, out0, out1,
             acc, src_st, dst_st, rowsbuf, gsem, ssem):
    c = lax.axis_index("c")
    t = lax.axis_index("s")

    zb = t * TPT
    pltpu.sync_copy(zer.at[pl.ds(zb, TPT)], acc.at[pl.ds(zb, TPT)])
    plsc.subcore_barrier()

    if round0:
      ppr = CPR // 16
      cbase = c * (CPR // 2)
    else:
      ppr = CPR // 8
      cbase = c * 0
    npairs = 2 * ppr
    ngroups = 2 * npairs
    niter = ngroups // 8

    def stage(P, ss):
      row = 2 * t + P // ppr
      coff = cbase + (P % ppr) * 8
      pltpu.async_copy(srcp.at[row, pl.ds(coff, 8)], src_st.at[ss],
                       ssem.at[ss, 0])
      pltpu.async_copy(dstp.at[row, pl.ds(coff, 8)], dst_st.at[ss],
                       ssem.at[ss, 1])

    def wait_stage(ss):
      pltpu.make_async_copy(srcp.at[0, pl.ds(0, 8)], src_st.at[ss],
                            ssem.at[ss, 0]).wait()
      pltpu.make_async_copy(dstp.at[0, pl.ds(0, 8)], dst_st.at[ss],
                            ssem.at[ss, 1]).wait()

    def run_pipeline(table):
      def fire(gs, ss, half):
        for j in range(4):
          pltpu.async_copy(table.at[src_st.at[ss, half * 4 + j]],
                           rowsbuf.at[gs, j], gsem.at[gs, j])

      def drain(gs, ss, half):
        for j in range(4):
          pltpu.make_async_copy(table.at[src_st.at[ss, half * 4 + j]],
                                rowsbuf.at[gs, j], gsem.at[gs, j]).wait()
          pltpu.sync_copy(rowsbuf.at[gs, j], acc.at[dst_st.at[ss, half * 4 + j]],
                          add=True)

      for p in range(4):
        stage(jnp.int32(p), p)

      def loop_body(m, carry):
        for cc in range(8):
          ss = cc // 2
          half = cc % 2
          gs = cc % 2
          if half == 0:
            wait_stage(ss)
          fire(gs, ss, half)
          pgs = (cc + 1) % 2
          pss = ((cc - 1) % 8) // 2
          phalf = (cc - 1) % 2
          if cc == 0:
            @pl.when(m > 0)
            def _():
              drain(pgs, pss, phalf)

            @pl.when(m > 0)
            def _():
              stage(m * 4 + 3, 3)
          else:
            drain(pgs, pss, phalf)
            if cc in (2, 4, 6):
              rs = cc // 2 - 1
              P = m * 4 + 4 + rs

              @pl.when(P < npairs)
              def _():
                stage(P, rs)
        return carry

      lax.fori_loop(0, niter, loop_body, jnp.int32(0))
      drain(1, 3, 1)

    @pl.when(c == 0)
    def _():
      run_pipeline(t0)

    @pl.when(c == 1)
    def _():
      run_pipeline(t1)

    plsc.subcore_barrier()

    ob = t * WB0

    @pl.when(t < 15)
    def _():
      @pl.when(c == 0)
      def _():
        pltpu.sync_copy(acc.at[pl.ds(ob, WB0)], out0.at[pl.ds(ob, WB0)])

      @pl.when(c == 1)
      def _():
        pltpu.sync_copy(acc.at[pl.ds(ob, WB0)], out1.at[pl.ds(ob, WB0)])

    @pl.when(t == 15)
    def _():
      @pl.when(c == 0)
      def _():
        pltpu.sync_copy(acc.at[pl.ds(15 * WB0, WB15)],
                        out0.at[pl.ds(15 * WB0, WB15)])

      @pl.when(c == 1)
      def _():
        pltpu.sync_copy(acc.at[pl.ds(15 * WB0, WB15)],
                        out1.at[pl.ds(15 * WB0, WB15)])

  return sc_agg


_sc_agg_round0 = _make_sc_aggregate(True)
_sc_agg_rounds = _make_sc_aggregate(False)

_BLK = 2000
_NBLK = N // _BLK


def _dot_hi(a, b):
  return jax.lax.dot_general(a, b, (((1,), (0,)), ((), ())),
                             precision=jax.lax.Precision.HIGHEST,
                             preferred_element_type=_F32)


def _dot_def(a, b):
  return jax.lax.dot_general(a, b, (((1,), (0,)), ((), ())),
                             precision=jax.lax.Precision.DEFAULT,
                             preferred_element_type=_F32)


def _rt16(v):
  return v.astype(jnp.bfloat16).astype(_F32)


_B16 = _BLK // 8
_PSH = (_NBLK, _B16, 128)


def _dspec():
  return pl.BlockSpec((1, _B16, 128), lambda i: (i, 0, 0))


def _wspec(r, c):
  return pl.BlockSpec((r, c), lambda i: (0, 0))


def _tc_combine0(x16p, a0, a1, bws_lo, bws_hi, bwn_lo, bwn_hi, blo, bhi):
  def body(x_ref, a0_ref, a1_ref, wsl_ref, wsh_ref, wnl_ref, wnh_ref,
           bl_ref, bh_ref, hlo_ref, hhi_ref, glo_ref, ghi_ref):
    x = x_ref[0]
    agg = a0_ref[0] + a1_ref[0]
    hlo = jnp.maximum(_dot_def(x, wsl_ref[...]) + _dot_hi(agg, wnl_ref[...])
                      + bl_ref[...], 0.0)
    hhi = jnp.maximum(_dot_def(x, wsh_ref[...]) + _dot_hi(agg, wnh_ref[...])
                      + bh_ref[...], 0.0)
    hlo_ref[0] = hlo
    hhi_ref[0] = hhi
    glo_ref[0] = hlo.astype(jnp.bfloat16).astype(_F32)
    ghi_ref[0] = hhi.astype(jnp.bfloat16).astype(_F32)

  return pl.pallas_call(
      body,
      grid=(_NBLK,),
      in_specs=[_dspec(), _dspec(), _dspec(),
                _wspec(128, 128), _wspec(128, 128),
                _wspec(128, 128), _wspec(128, 128),
                _wspec(1, 128), _wspec(1, 128)],
      out_specs=[_dspec(), _dspec(), _dspec(), _dspec()],
      out_shape=[jax.ShapeDtypeStruct(_PSH, _F32)] * 4,
  )(x16p, a0, a1, bws_lo, bws_hi, bwn_lo, bwn_hi, blo, bhi)


def _tc_res(hlo_p, hhi_p, alo, ahi, bw_ll, bw_hl, bw_lh, bw_hh, blo, bhi):
  def body(hl_ref, hh_ref, al_ref, ah_ref, wll_ref, whl_ref, wlh_ref,
           whh_ref, bl_ref, bh_ref, olo_ref, ohi_ref, glo_ref, ghi_ref):
    al = al_ref[0]
    ah = ah_ref[0]
    hlo = jnp.maximum(hl_ref[0] + _dot_hi(al, wll_ref[...])
                      + _dot_hi(ah, whl_ref[...]) + bl_ref[...], 0.0)
    hhi = jnp.maximum(hh_ref[0] + _dot_hi(al, wlh_ref[...])
                      + _dot_hi(ah, whh_ref[...]) + bh_ref[...], 0.0)
    olo_ref[0] = hlo
    ohi_ref[0] = hhi
    glo_ref[0] = hlo.astype(jnp.bfloat16).astype(_F32)
    ghi_ref[0] = hhi.astype(jnp.bfloat16).astype(_F32)

  return pl.pallas_call(
      body,
      grid=(_NBLK,),
      in_specs=[_dspec(), _dspec(), _dspec(), _dspec(),
                _wspec(128, 128), _wspec(128, 128),
                _wspec(128, 128), _wspec(128, 128),
                _wspec(1, 128), _wspec(1, 128)],
      out_specs=[_dspec(), _dspec(), _dspec(), _dspec()],
      out_shape=[jax.ShapeDtypeStruct(_PSH, _F32)] * 4,
  )(hlo_p, hhi_p, alo, ahi, bw_ll, bw_hl, bw_lh, bw_hh, blo, bhi)


def _tc_final(hlo_p, hhi_p, alo, ahi, bw_ll, bw_hl, bw_lh, bw_hh, blo, bhi,
              bw1_lo, bw1_hi, b1t, bw2, bfc2):
  def body(hl_ref, hh_ref, al_ref, ah_ref, wll_ref, whl_ref, wlh_ref,
           whh_ref, bl_ref, bh_ref, w1l_ref, w1h_ref, b1_ref, w2_ref,
           b2_ref, out_ref):
    i = pl.program_id(0)
    al = al_ref[0]
    ah = ah_ref[0]
    h3lo = jnp.maximum(hl_ref[0] + _dot_hi(al, wll_ref[...])
                       + _dot_hi(ah, whl_ref[...]) + bl_ref[...], 0.0)
    h3hi = jnp.maximum(hh_ref[0] + _dot_hi(al, wlh_ref[...])
                       + _dot_hi(ah, whh_ref[...]) + bh_ref[...], 0.0)
    z = jnp.maximum(_dot_def(h3lo, w1l_ref[...])
                    + _dot_def(h3hi, w1h_ref[...]) + b1_ref[...], 0.0)
    s = jnp.sum(_dot_def(z, w2_ref[...]))

    @pl.when(i == 0)
    def _():
      out_ref[...] = jnp.zeros((1, 1), _F32)

    out_ref[...] += s.reshape(1, 1)

    @pl.when(i == _NBLK - 1)
    def _():
      out_ref[...] = out_ref[...] / N + b2_ref[...]

  return pl.pallas_call(
      body,
      grid=(_NBLK,),
      in_specs=[_dspec(), _dspec(), _dspec(), _dspec(),
                _wspec(128, 128), _wspec(128, 128),
                _wspec(128, 128), _wspec(128, 128),
                _wspec(1, 128), _wspec(1, 128),
                _wspec(128, 240), _wspec(128, 240),
                _wspec(1, 240), _wspec(240, 8), _wspec(1, 1)],
      out_specs=pl.BlockSpec((1, 1), lambda i: (0, 0)),
      out_shape=jax.ShapeDtypeStruct((1, 1), _F32),
  )(hlo_p, hhi_p, alo, ahi, bw_ll, bw_hl, bw_lh, bw_hh, blo, bhi,
    bw1_lo, bw1_hi, b1t, bw2, bfc2)


def kernel(x, edge_index, w_conv_self, w_conv_nbr, b_conv, w_res, b_res,
           w_fc1, b_fc1, w_fc2, b_fc2):
  src = edge_index[0].reshape(NROWS, EPR)
  dst = edge_index[1].reshape(NROWS, EPR)
  srcp = jnp.concatenate([src, jnp.asarray(_PAD_SRC)],
                         axis=1).reshape(NROWS, CPR, 128)
  dstp = jnp.concatenate([dst, jnp.asarray(_PAD_DST)],
                         axis=1).reshape(NROWS, CPR, 128)

  eye8 = jnp.eye(8, dtype=_F32)
  bd = lambda w: jnp.kron(eye8, w)

  x16 = jnp.concatenate([x, jnp.zeros((N, 9), _F32)], axis=1)
  x16p = x16.reshape(_PSH)
  ws16 = jnp.concatenate([w_conv_self, jnp.zeros((9, NF), _F32)], axis=0)
  wn16r = _rt16(jnp.concatenate([w_conv_nbr, jnp.zeros((9, NF), _F32)],
                                axis=0))
  zer = jnp.zeros((NACC, 16), _F32)

  xg = _rt16(x16p).reshape(N, 16)
  a0, a1 = _sc_agg_round0(xg, xg, srcp, dstp, zer)
  hlo_p, hhi_p, glo, ghi = _tc_combine0(
      x16p, a0.reshape(_PSH), a1.reshape(_PSH),
      bd(ws16[:, :16]), bd(ws16[:, 16:]),
      bd(wn16r[:, :16]), bd(wn16r[:, 16:]),
      jnp.tile(b_conv[:16], 8).reshape(1, 128),
      jnp.tile(b_conv[16:], 8).reshape(1, 128))

  for i in range(3):
    alo, ahi = _sc_agg_rounds(glo.reshape(N, 16), ghi.reshape(N, 16),
                              srcp, dstp, zer)
    wr = _rt16(w_res[i])
    bws = (bd(wr[:16, :16]), bd(wr[16:, :16]),
           bd(wr[:16, 16:]), bd(wr[16:, 16:]))
    brs = (jnp.tile(b_res[i, :16], 8).reshape(1, 128),
           jnp.tile(b_res[i, 16:], 8).reshape(1, 128))
    if i < 2:
      hlo_p, hhi_p, glo, ghi = _tc_res(hlo_p, hhi_p, alo.reshape(_PSH),
                                       ahi.reshape(_PSH), *bws, *brs)
    else:
      out = _tc_final(hlo_p, hhi_p, alo.reshape(_PSH), ahi.reshape(_PSH),
                      *bws, *brs,
                      bd(w_fc1[:16, :]), bd(w_fc1[16:, :]),
                      jnp.tile(b_fc1, 8).reshape(1, 240),
                      bd(w_fc2), b_fc2.reshape(1, 1))
  return out[0, 0]


# speedup vs baseline: 29.0307x; 1.1020x over previous
import functools

import jax
import jax.numpy as jnp
import numpy as np
from jax import lax
from jax.experimental import pallas as pl
from jax.experimental.pallas import tpu as pltpu
from jax.experimental.pallas import tpu_sc as plsc

N = 100000
E = 1600000
NF = 32

NROWS = 32
EPR = E // NROWS
CPR = 416
RPAD = CPR * 128 - EPR
NACC = N + 96
TPT = NACC // 16
WB0 = 6256
WB15 = N - 15 * WB0

_rr, _cc = np.meshgrid(np.arange(NROWS), np.arange(RPAD), indexing="ij")
_PAD_SRC = np.asarray((_rr * 1009 + _cc * 257) % N, dtype=np.int32)
_PAD_DST = np.asarray(N + (_cc % 96), dtype=np.int32)

_MESH = plsc.VectorSubcoreMesh(
    core_axis_name="c", subcore_axis_name="s", num_cores=2, num_subcores=16
)

_F32 = jnp.float32


def _make_sc_aggregate(round0: bool):
  @functools.partial(
      pl.kernel,
      out_type=[jax.ShapeDtypeStruct((N, 16), _F32),
                jax.ShapeDtypeStruct((N, 16), _F32)],
      mesh=_MESH,
      scratch_types=[
          pltpu.VMEM_SHARED((NACC, 16), _F32),
          pltpu.VMEM((4, 8, 128), jnp.int32),
          pltpu.VMEM((4, 8, 128), jnp.int32),
          pltpu.VMEM((2, 4, 128, 16), _F32),
          pltpu.SemaphoreType.DMA((2, 4)),
          pltpu.SemaphoreType.DMA((4, 2)),
          pltpu.SemaphoreType.DMA((2, 4)),
      ],
      compiler_params=pltpu.CompilerParams(use_tc_tiling_on_sc=False),
  )
  def sc_agg(t0, t1, srcp, dstp, zer, out0, out1,
             acc, src_st, dst_st, rowsbuf, gsem, ssem, scsem):
    c = lax.axis_index("c")
    t = lax.axis_index("s")

    zb = t * TPT
    pltpu.sync_copy(zer.at[pl.ds(zb, TPT)], acc.at[pl.ds(zb, TPT)])
    plsc.subcore_barrier()

    if round0:
      ppr = CPR // 16
      cbase = c * (CPR // 2)
    else:
      ppr = CPR // 8
      cbase = c * 0
    npairs = 2 * ppr
    ngroups = 2 * npairs
    niter = ngroups // 8

    def stage(P, ss):
      row = 2 * t + P // ppr
      coff = cbase + (P % ppr) * 8
      pltpu.async_copy(srcp.at[row, pl.ds(coff, 8)], src_st.at[ss],
                       ssem.at[ss, 0])
      pltpu.async_copy(dstp.at[row, pl.ds(coff, 8)], dst_st.at[ss],
                       ssem.at[ss, 1])

    def wait_stage(ss):
      pltpu.make_async_copy(srcp.at[0, pl.ds(0, 8)], src_st.at[ss],
                            ssem.at[ss, 0]).wait()
      pltpu.make_async_copy(dstp.at[0, pl.ds(0, 8)], dst_st.at[ss],
                            ssem.at[ss, 1]).wait()

    def run_pipeline(table):
      def wait_scatter(gs, j):
        pltpu.make_async_copy(rowsbuf.at[gs, j], acc.at[dst_st.at[0, j]],
                              scsem.at[gs, j]).wait()

      def fire(gs, ss, half, guard=None):
        for j in range(4):
          if guard is None:
            wait_scatter(gs, j)
          else:
            @pl.when(guard)
            def _():
              wait_scatter(gs, j)
          pltpu.async_copy(table.at[src_st.at[ss, half * 4 + j]],
                           rowsbuf.at[gs, j], gsem.at[gs, j])

      def drain(gs, ss, half):
        for j in range(4):
          pltpu.make_async_copy(table.at[src_st.at[ss, half * 4 + j]],
                                rowsbuf.at[gs, j], gsem.at[gs, j]).wait()
          pltpu.async_copy(rowsbuf.at[gs, j],
                           acc.at[dst_st.at[ss, half * 4 + j]],
                           scsem.at[gs, j], add=True)

      for p in range(4):
        stage(jnp.int32(p), p)

      def loop_body(m, carry):
        for cc in range(8):
          ss = cc // 2
          half = cc % 2
          gs = cc % 2
          if half == 0:
            wait_stage(ss)
          fire(gs, ss, half, guard=(m > 0) if cc < 2 else None)
          pgs = (cc + 1) % 2
          pss = ((cc - 1) % 8) // 2
          phalf = (cc - 1) % 2
          if cc == 0:
            @pl.when(m > 0)
            def _():
              drain(pgs, pss, phalf)

            @pl.when(m > 0)
            def _():
              stage(m * 4 + 3, 3)
          else:
            drain(pgs, pss, phalf)
            if cc in (2, 4, 6):
              rs = cc // 2 - 1
              P = m * 4 + 4 + rs

              @pl.when(P < npairs)
              def _():
                stage(P, rs)
        return carry

      lax.fori_loop(0, niter, loop_body, jnp.int32(0))
      drain(1, 3, 1)
      for gs in range(2):
        for j in range(4):
          wait_scatter(gs, j)

    @pl.when(c == 0)
    def _():
      run_pipeline(t0)

    @pl.when(c == 1)
    def _():
      run_pipeline(t1)

    plsc.subcore_barrier()

    ob = t * WB0

    @pl.when(t < 15)
    def _():
      @pl.when(c == 0)
      def _():
        pltpu.sync_copy(acc.at[pl.ds(ob, WB0)], out0.at[pl.ds(ob, WB0)])

      @pl.when(c == 1)
      def _():
        pltpu.sync_copy(acc.at[pl.ds(ob, WB0)], out1.at[pl.ds(ob, WB0)])

    @pl.when(t == 15)
    def _():
      @pl.when(c == 0)
      def _():
        pltpu.sync_copy(acc.at[pl.ds(15 * WB0, WB15)],
                        out0.at[pl.ds(15 * WB0, WB15)])

      @pl.when(c == 1)
      def _():
        pltpu.sync_copy(acc.at[pl.ds(15 * WB0, WB15)],
                        out1.at[pl.ds(15 * WB0, WB15)])

  return sc_agg


_sc_agg_round0 = _make_sc_aggregate(True)
_sc_agg_rounds = _make_sc_aggregate(False)

_BLK = 2000
_NBLK = N // _BLK


def _dot_hi(a, b):
  return jax.lax.dot_general(a, b, (((1,), (0,)), ((), ())),
                             precision=jax.lax.Precision.HIGHEST,
                             preferred_element_type=_F32)


def _dot_def(a, b):
  return jax.lax.dot_general(a, b, (((1,), (0,)), ((), ())),
                             precision=jax.lax.Precision.DEFAULT,
                             preferred_element_type=_F32)


def _rt16(v):
  return v.astype(jnp.bfloat16).astype(_F32)


_B16 = _BLK // 8
_PSH = (_NBLK, _B16, 128)


def _dspec():
  return pl.BlockSpec((1, _B16, 128), lambda i: (i, 0, 0))


def _wspec(r, c):
  return pl.BlockSpec((r, c), lambda i: (0, 0))


def _tc_combine0(x16p, a0, a1, bws_lo, bws_hi, bwn_lo, bwn_hi, blo, bhi):
  def body(x_ref, a0_ref, a1_ref, wsl_ref, wsh_ref, wnl_ref, wnh_ref,
           bl_ref, bh_ref, hlo_ref, hhi_ref, glo_ref, ghi_ref):
    x = x_ref[0]
    agg = a0_ref[0] + a1_ref[0]
    hlo = jnp.maximum(_dot_def(x, wsl_ref[...]) + _dot_hi(agg, wnl_ref[...])
                      + bl_ref[...], 0.0)
    hhi = jnp.maximum(_dot_def(x, wsh_ref[...]) + _dot_hi(agg, wnh_ref[...])
                      + bh_ref[...], 0.0)
    hlo_ref[0] = hlo
    hhi_ref[0] = hhi
    glo_ref[0] = hlo.astype(jnp.bfloat16).astype(_F32)
    ghi_ref[0] = hhi.astype(jnp.bfloat16).astype(_F32)

  return pl.pallas_call(
      body,
      grid=(_NBLK,),
      in_specs=[_dspec(), _dspec(), _dspec(),
                _wspec(128, 128), _wspec(128, 128),
                _wspec(128, 128), _wspec(128, 128),
                _wspec(1, 128), _wspec(1, 128)],
      out_specs=[_dspec(), _dspec(), _dspec(), _dspec()],
      out_shape=[jax.ShapeDtypeStruct(_PSH, _F32)] * 4,
  )(x16p, a0, a1, bws_lo, bws_hi, bwn_lo, bwn_hi, blo, bhi)


def _tc_res(hlo_p, hhi_p, alo, ahi, bw_ll, bw_hl, bw_lh, bw_hh, blo, bhi):
  def body(hl_ref, hh_ref, al_ref, ah_ref, wll_ref, whl_ref, wlh_ref,
           whh_ref, bl_ref, bh_ref, olo_ref, ohi_ref, glo_ref, ghi_ref):
    al = al_ref[0]
    ah = ah_ref[0]
    hlo = jnp.maximum(hl_ref[0] + _dot_hi(al, wll_ref[...])
                      + _dot_hi(ah, whl_ref[...]) + bl_ref[...], 0.0)
    hhi = jnp.maximum(hh_ref[0] + _dot_hi(al, wlh_ref[...])
                      + _dot_hi(ah, whh_ref[...]) + bh_ref[...], 0.0)
    olo_ref[0] = hlo
    ohi_ref[0] = hhi
    glo_ref[0] = hlo.astype(jnp.bfloat16).astype(_F32)
    ghi_ref[0] = hhi.astype(jnp.bfloat16).astype(_F32)

  return pl.pallas_call(
      body,
      grid=(_NBLK,),
      in_specs=[_dspec(), _dspec(), _dspec(), _dspec(),
                _wspec(128, 128), _wspec(128, 128),
                _wspec(128, 128), _wspec(128, 128),
                _wspec(1, 128), _wspec(1, 128)],
      out_specs=[_dspec(), _dspec(), _dspec(), _dspec()],
      out_shape=[jax.ShapeDtypeStruct(_PSH, _F32)] * 4,
  )(hlo_p, hhi_p, alo, ahi, bw_ll, bw_hl, bw_lh, bw_hh, blo, bhi)


def _tc_final(hlo_p, hhi_p, alo, ahi, bw_ll, bw_hl, bw_lh, bw_hh, blo, bhi,
              bw1_lo, bw1_hi, b1t, bw2, bfc2):
  def body(hl_ref, hh_ref, al_ref, ah_ref, wll_ref, whl_ref, wlh_ref,
           whh_ref, bl_ref, bh_ref, w1l_ref, w1h_ref, b1_ref, w2_ref,
           b2_ref, out_ref):
    i = pl.program_id(0)
    al = al_ref[0]
    ah = ah_ref[0]
    h3lo = jnp.maximum(hl_ref[0] + _dot_hi(al, wll_ref[...])
                       + _dot_hi(ah, whl_ref[...]) + bl_ref[...], 0.0)
    h3hi = jnp.maximum(hh_ref[0] + _dot_hi(al, wlh_ref[...])
                       + _dot_hi(ah, whh_ref[...]) + bh_ref[...], 0.0)
    z = jnp.maximum(_dot_def(h3lo, w1l_ref[...])
                    + _dot_def(h3hi, w1h_ref[...]) + b1_ref[...], 0.0)
    s = jnp.sum(_dot_def(z, w2_ref[...]))

    @pl.when(i == 0)
    def _():
      out_ref[...] = jnp.zeros((1, 1), _F32)

    out_ref[...] += s.reshape(1, 1)

    @pl.when(i == _NBLK - 1)
    def _():
      out_ref[...] = out_ref[...] / N + b2_ref[...]

  return pl.pallas_call(
      body,
      grid=(_NBLK,),
      in_specs=[_dspec(), _dspec(), _dspec(), _dspec(),
                _wspec(128, 128), _wspec(128, 128),
                _wspec(128, 128), _wspec(128, 128),
                _wspec(1, 128), _wspec(1, 128),
                _wspec(128, 240), _wspec(128, 240),
                _wspec(1, 240), _wspec(240, 8), _wspec(1, 1)],
      out_specs=pl.BlockSpec((1, 1), lambda i: (0, 0)),
      out_shape=jax.ShapeDtypeStruct((1, 1), _F32),
  )(hlo_p, hhi_p, alo, ahi, bw_ll, bw_hl, bw_lh, bw_hh, blo, bhi,
    bw1_lo, bw1_hi, b1t, bw2, bfc2)


def kernel(x, edge_index, w_conv_self, w_conv_nbr, b_conv, w_res, b_res,
           w_fc1, b_fc1, w_fc2, b_fc2):
  src = edge_index[0].reshape(NROWS, EPR)
  dst = edge_index[1].reshape(NROWS, EPR)
  srcp = jnp.concatenate([src, jnp.asarray(_PAD_SRC)],
                         axis=1).reshape(NROWS, CPR, 128)
  dstp = jnp.concatenate([dst, jnp.asarray(_PAD_DST)],
                         axis=1).reshape(NROWS, CPR, 128)

  eye8 = jnp.eye(8, dtype=_F32)
  bd = lambda w: jnp.kron(eye8, w)

  x16 = jnp.concatenate([x, jnp.zeros((N, 9), _F32)], axis=1)
  x16p = x16.reshape(_PSH)
  ws16 = jnp.concatenate([w_conv_self, jnp.zeros((9, NF), _F32)], axis=0)
  wn16r = _rt16(jnp.concatenate([w_conv_nbr, jnp.zeros((9, NF), _F32)],
                                axis=0))
  zer = jnp.zeros((NACC, 16), _F32)

  xg = _rt16(x16p).reshape(N, 16)
  a0, a1 = _sc_agg_round0(xg, xg, srcp, dstp, zer)
  hlo_p, hhi_p, glo, ghi = _tc_combine0(
      x16p, a0.reshape(_PSH), a1.reshape(_PSH),
      bd(ws16[:, :16]), bd(ws16[:, 16:]),
      bd(wn16r[:, :16]), bd(wn16r[:, 16:]),
      jnp.tile(b_conv[:16], 8).reshape(1, 128),
      jnp.tile(b_conv[16:], 8).reshape(1, 128))

  for i in range(3):
    alo, ahi = _sc_agg_rounds(glo.reshape(N, 16), ghi.reshape(N, 16),
                              srcp, dstp, zer)
    wr = _rt16(w_res[i])
    bws = (bd(wr[:16, :16]), bd(wr[16:, :16]),
           bd(wr[:16, 16:]), bd(wr[16:, 16:]))
    brs = (jnp.tile(b_res[i, :16], 8).reshape(1, 128),
           jnp.tile(b_res[i, 16:], 8).reshape(1, 128))
    if i < 2:
      hlo_p, hhi_p, glo, ghi = _tc_res(hlo_p, hhi_p, alo.reshape(_PSH),
                                       ahi.reshape(_PSH), *bws, *brs)
    else:
      out = _tc_final(hlo_p, hhi_p, alo.reshape(_PSH), ahi.reshape(_PSH),
                      *bws, *brs,
                      bd(w_fc1[:16, :]), bd(w_fc1[16:, :]),
                      jnp.tile(b_fc1, 8).reshape(1, 240),
                      bd(w_fc2), b_fc2.reshape(1, 1))
  return out[0, 0]
